# baseline (device time: 2342108 ns/iter reference)
import jax
import jax.numpy as jnp
from jax import lax
from jax.experimental import pallas as pl
from jax.experimental.pallas import tpu as pltpu

NZ = 4
S = 8


def _passthrough(y):
    m, n = y.shape
    nchunks = 8
    rows = m // nchunks

    def body(y_ref, o_ref, sems):
        for i in range(nchunks):
            pltpu.make_async_copy(
                y_ref.at[pl.ds(i * rows, rows), :],
                o_ref.at[pl.ds(i * rows, rows), :],
                sems.at[i],
            ).start()
        for i in range(nchunks):
            pltpu.make_async_copy(
                y_ref.at[pl.ds(i * rows, rows), :],
                o_ref.at[pl.ds(i * rows, rows), :],
                sems.at[i],
            ).wait()

    return pl.pallas_call(
        body,
        out_shape=jax.ShapeDtypeStruct((m, n), y.dtype),
        in_specs=[pl.BlockSpec(memory_space=pl.ANY)],
        out_specs=pl.BlockSpec(memory_space=pl.ANY),
        scratch_shapes=[pltpu.SemaphoreType.DMA((nchunks,))],
    )(y)


def kernel(x):
    x = x.astype(jnp.bfloat16)
    m_per, n = x.shape
    half = m_per // 2
    seg = half // S

    def body(x_ref, out_ref, copy_sem,
             szr, szl, rzr, rzl, sxr, sxl, rxr, rxl):
        my_x = lax.axis_index("x")
        my_y = lax.axis_index("y")
        my_z = lax.axis_index("z")

        right = (my_x, my_y, my_z + 1)
        left = (my_x, my_y, my_z - 1)
        partner = (1 - my_x, my_y, my_z)

        has_right = my_z < NZ - 1
        has_left = my_z > 0

        def sl(c, s_, parity):
            return out_ref.at[pl.ds(c * m_per + parity * half + s_ * seg, seg), :]

        def send(src, dst, ssem, rsem, dev):
            pltpu.make_async_remote_copy(
                src_ref=src, dst_ref=dst, send_sem=ssem, recv_sem=rsem,
                device_id=dev, device_id_type=pl.DeviceIdType.MESH,
            ).start()

        def wait_recv(dst, rsem):
            pltpu.make_async_remote_copy(
                src_ref=dst, dst_ref=dst, send_sem=copy_sem, recv_sem=rsem,
                device_id=partner, device_id_type=pl.DeviceIdType.MESH,
            ).wait_recv()

        def wait_send(src, ssem):
            pltpu.make_async_remote_copy(
                src_ref=src, dst_ref=src, send_sem=ssem, recv_sem=copy_sem,
                device_id=partner, device_id_type=pl.DeviceIdType.MESH,
            ).wait_send()

        barrier_sem = pltpu.get_barrier_semaphore()
        pl.semaphore_signal(barrier_sem, inc=1, device_id=partner,
                            device_id_type=pl.DeviceIdType.MESH)

        @pl.when(has_right)
        def _():
            pl.semaphore_signal(barrier_sem, inc=1, device_id=right,
                                device_id_type=pl.DeviceIdType.MESH)

        @pl.when(has_left)
        def _():
            pl.semaphore_signal(barrier_sem, inc=1, device_id=left,
                                device_id_type=pl.DeviceIdType.MESH)

        n_nbrs = 1 + has_right.astype(jnp.int32) + has_left.astype(jnp.int32)
        pl.semaphore_wait(barrier_sem, n_nbrs)

        cp = pltpu.make_async_copy(
            x_ref, out_ref.at[pl.ds(my_z * m_per, m_per), :], copy_sem
        )
        cp.start()

        for s_ in range(S):
            src = x_ref.at[pl.ds(my_x * half + s_ * seg, seg), :]

            @pl.when(has_right)
            def _():
                send(src, sl(my_z, s_, my_x), szr.at[0, s_], rzr.at[0, s_], right)

            @pl.when(has_left)
            def _():
                send(src, sl(my_z, s_, my_x), szl.at[0, s_], rzl.at[0, s_], left)

        for idx in range(NZ - 1):
            for s_ in range(S):
                r_ev = idx < my_z
                l_ev = idx < NZ - 1 - my_z

                @pl.when(r_ev)
                def _():
                    c = my_z - 1 - idx
                    dst = sl(c, s_, my_x)
                    wait_recv(dst, rzr.at[idx, s_])

                    @pl.when(has_right)
                    def _():
                        send(dst, dst, szr.at[idx + 1, s_],
                             rzr.at[idx + 1, s_], right)

                    send(dst, dst, sxr.at[idx, s_], rxr.at[idx, s_], partner)

                @pl.when(l_ev)
                def _():
                    c = my_z + 1 + idx
                    dst = sl(c, s_, my_x)
                    wait_recv(dst, rzl.at[idx, s_])

                    @pl.when(has_left)
                    def _():
                        send(dst, dst, szl.at[idx + 1, s_],
                             rzl.at[idx + 1, s_], left)

                    send(dst, dst, sxl.at[idx, s_], rxl.at[idx, s_], partner)

        cp.wait()

        for idx in range(NZ - 1):
            for s_ in range(S):
                @pl.when(idx < my_z)
                def _():
                    wait_recv(sl(my_z - 1 - idx, s_, 1 - my_x), rxr.at[idx, s_])

                @pl.when(idx < NZ - 1 - my_z)
                def _():
                    wait_recv(sl(my_z + 1 + idx, s_, 1 - my_x), rxl.at[idx, s_])

        for idx in range(NZ - 1):
            for s_ in range(S):
                r_ev = idx < my_z
                l_ev = idx < NZ - 1 - my_z
                own_src = x_ref.at[pl.ds(my_x * half + s_ * seg, seg), :]

                @pl.when(has_right if idx == 0 else jnp.logical_and(has_right, idx - 1 < my_z))
                def _():
                    wait_send(own_src, szr.at[idx, s_])

                @pl.when(has_left if idx == 0 else jnp.logical_and(has_left, idx - 1 < NZ - 1 - my_z))
                def _():
                    wait_send(own_src, szl.at[idx, s_])

                @pl.when(r_ev)
                def _():
                    wait_send(own_src, sxr.at[idx, s_])

                @pl.when(l_ev)
                def _():
                    wait_send(own_src, sxl.at[idx, s_])

    sems3 = pltpu.SemaphoreType.DMA((NZ - 1, S))
    gathered = pl.pallas_call(
        body,
        out_shape=jax.ShapeDtypeStruct((NZ * m_per, n), jnp.bfloat16),
        in_specs=[pl.BlockSpec(memory_space=pl.ANY)],
        out_specs=pl.BlockSpec(memory_space=pltpu.MemorySpace.HBM),
        scratch_shapes=[
            pltpu.SemaphoreType.DMA,
            sems3, sems3,
            sems3, sems3,
            sems3, sems3,
            sems3, sems3,
        ],
        compiler_params=pltpu.CompilerParams(collective_id=0),
    )(x)
    return _passthrough(gathered)


# device time: 349818 ns/iter; 6.6952x vs baseline; 6.6952x over previous
import jax
import jax.numpy as jnp
from jax import lax
from jax.experimental import pallas as pl
from jax.experimental.pallas import tpu as pltpu

NZ = 4
S = 8


def _passthrough(y):
    m, n = y.shape
    T = 32
    D = 4
    rows = m // T

    def body(y_ref, o_ref, vbuf, isems, osems):
        def in_copy(t):
            b = t % D
            return pltpu.make_async_copy(
                y_ref.at[pl.ds(t * rows, rows), :], vbuf.at[b], isems.at[b]
            )

        def out_copy(t):
            b = t % D
            return pltpu.make_async_copy(
                vbuf.at[b], o_ref.at[pl.ds(t * rows, rows), :], osems.at[b]
            )

        for t in range(T + 1):
            if t < T:
                if t >= D:
                    out_copy(t - D).wait()
                in_copy(t).start()
            if t >= 1:
                in_copy(t - 1).wait()
                out_copy(t - 1).start()
        for t in range(T - D, T):
            out_copy(t).wait()

    return pl.pallas_call(
        body,
        out_shape=jax.ShapeDtypeStruct((m, n), y.dtype),
        in_specs=[pl.BlockSpec(memory_space=pl.ANY)],
        out_specs=pl.BlockSpec(memory_space=pl.ANY),
        scratch_shapes=[
            pltpu.VMEM((D, rows, n), y.dtype),
            pltpu.SemaphoreType.DMA((D,)),
            pltpu.SemaphoreType.DMA((D,)),
        ],
    )(y)


def kernel(x):
    x = x.astype(jnp.bfloat16)
    m_per, n = x.shape
    half = m_per // 2
    seg = half // S

    def body(x_ref, out_ref, copy_sem,
             szr, szl, rzr, rzl, sxr, sxl, rxr, rxl):
        my_x = lax.axis_index("x")
        my_y = lax.axis_index("y")
        my_z = lax.axis_index("z")

        right = (my_x, my_y, my_z + 1)
        left = (my_x, my_y, my_z - 1)
        partner = (1 - my_x, my_y, my_z)

        has_right = my_z < NZ - 1
        has_left = my_z > 0

        def sl(c, s_, parity):
            return out_ref.at[pl.ds(c * m_per + parity * half + s_ * seg, seg), :]

        def send(src, dst, ssem, rsem, dev):
            pltpu.make_async_remote_copy(
                src_ref=src, dst_ref=dst, send_sem=ssem, recv_sem=rsem,
                device_id=dev, device_id_type=pl.DeviceIdType.MESH,
            ).start()

        def wait_recv(dst, rsem):
            pltpu.make_async_remote_copy(
                src_ref=dst, dst_ref=dst, send_sem=copy_sem, recv_sem=rsem,
                device_id=partner, device_id_type=pl.DeviceIdType.MESH,
            ).wait_recv()

        def wait_send(src, ssem):
            pltpu.make_async_remote_copy(
                src_ref=src, dst_ref=src, send_sem=ssem, recv_sem=copy_sem,
                device_id=partner, device_id_type=pl.DeviceIdType.MESH,
            ).wait_send()

        barrier_sem = pltpu.get_barrier_semaphore()
        pl.semaphore_signal(barrier_sem, inc=1, device_id=partner,
                            device_id_type=pl.DeviceIdType.MESH)

        @pl.when(has_right)
        def _():
            pl.semaphore_signal(barrier_sem, inc=1, device_id=right,
                                device_id_type=pl.DeviceIdType.MESH)

        @pl.when(has_left)
        def _():
            pl.semaphore_signal(barrier_sem, inc=1, device_id=left,
                                device_id_type=pl.DeviceIdType.MESH)

        n_nbrs = 1 + has_right.astype(jnp.int32) + has_left.astype(jnp.int32)
        pl.semaphore_wait(barrier_sem, n_nbrs)

        cp = pltpu.make_async_copy(
            x_ref, out_ref.at[pl.ds(my_z * m_per, m_per), :], copy_sem
        )
        cp.start()

        for s_ in range(S):
            src = x_ref.at[pl.ds(my_x * half + s_ * seg, seg), :]

            @pl.when(has_right)
            def _():
                send(src, sl(my_z, s_, my_x), szr.at[0, s_], rzr.at[0, s_], right)

            @pl.when(has_left)
            def _():
                send(src, sl(my_z, s_, my_x), szl.at[0, s_], rzl.at[0, s_], left)

        for idx in range(NZ - 1):
            for s_ in range(S):
                r_ev = idx < my_z
                l_ev = idx < NZ - 1 - my_z

                @pl.when(r_ev)
                def _():
                    c = my_z - 1 - idx
                    dst = sl(c, s_, my_x)
                    wait_recv(dst, rzr.at[idx, s_])

                    @pl.when(has_right)
                    def _():
                        send(dst, dst, szr.at[idx + 1, s_],
                             rzr.at[idx + 1, s_], right)

                    send(dst, dst, sxr.at[idx, s_], rxr.at[idx, s_], partner)

                @pl.when(l_ev)
                def _():
                    c = my_z + 1 + idx
                    dst = sl(c, s_, my_x)
                    wait_recv(dst, rzl.at[idx, s_])

                    @pl.when(has_left)
                    def _():
                        send(dst, dst, szl.at[idx + 1, s_],
                             rzl.at[idx + 1, s_], left)

                    send(dst, dst, sxl.at[idx, s_], rxl.at[idx, s_], partner)

        cp.wait()

        for idx in range(NZ - 1):
            for s_ in range(S):
                @pl.when(idx < my_z)
                def _():
                    wait_recv(sl(my_z - 1 - idx, s_, 1 - my_x), rxr.at[idx, s_])

                @pl.when(idx < NZ - 1 - my_z)
                def _():
                    wait_recv(sl(my_z + 1 + idx, s_, 1 - my_x), rxl.at[idx, s_])

        for idx in range(NZ - 1):
            for s_ in range(S):
                r_ev = idx < my_z
                l_ev = idx < NZ - 1 - my_z
                own_src = x_ref.at[pl.ds(my_x * half + s_ * seg, seg), :]

                @pl.when(has_right if idx == 0 else jnp.logical_and(has_right, idx - 1 < my_z))
                def _():
                    wait_send(own_src, szr.at[idx, s_])

                @pl.when(has_left if idx == 0 else jnp.logical_and(has_left, idx - 1 < NZ - 1 - my_z))
                def _():
                    wait_send(own_src, szl.at[idx, s_])

                @pl.when(r_ev)
                def _():
                    wait_send(own_src, sxr.at[idx, s_])

                @pl.when(l_ev)
                def _():
                    wait_send(own_src, sxl.at[idx, s_])

    sems3 = pltpu.SemaphoreType.DMA((NZ - 1, S))
    gathered = pl.pallas_call(
        body,
        out_shape=jax.ShapeDtypeStruct((NZ * m_per, n), jnp.bfloat16),
        in_specs=[pl.BlockSpec(memory_space=pl.ANY)],
        out_specs=pl.BlockSpec(memory_space=pltpu.MemorySpace.HBM),
        scratch_shapes=[
            pltpu.SemaphoreType.DMA,
            sems3, sems3,
            sems3, sems3,
            sems3, sems3,
            sems3, sems3,
        ],
        compiler_params=pltpu.CompilerParams(collective_id=0),
    )(x)
    return _passthrough(gathered)


# device time: 278847 ns/iter; 8.3993x vs baseline; 1.2545x over previous
import jax
import jax.numpy as jnp
from jax import lax
from jax.experimental import pallas as pl
from jax.experimental.pallas import tpu as pltpu

NZ = 4
S = 4


def kernel(x):
    x = x.astype(jnp.bfloat16)
    m_per, n = x.shape
    qrows = m_per // 4
    qseg = qrows // S

    def body(x_ref, out_ref, copy_sem,
             szr, szl, rzr, rzl,
             sxd, rxd, syd, ryd,
             sxrel, rxrel, syrel, ryrel):
        my_x = lax.axis_index("x")
        my_y = lax.axis_index("y")
        my_z = lax.axis_index("z")

        right = (my_x, my_y, my_z + 1)
        left = (my_x, my_y, my_z - 1)
        partner = (1 - my_x, my_y, my_z)
        buddy = (my_x, my_y ^ 1, my_z)

        has_right = my_z < NZ - 1
        has_left = my_z > 0

        q = 2 * (my_y % 2) + my_x
        qp = 2 * (my_y % 2) + (1 - my_x)
        qb = 2 * ((my_y ^ 1) % 2) + my_x
        qd = 2 * ((my_y ^ 1) % 2) + (1 - my_x)

        def qsl(c, qq, s_):
            return out_ref.at[pl.ds(c * m_per + qq * qrows + s_ * qseg, qseg), :]

        def send(src, dst, ssem, rsem, dev):
            pltpu.make_async_remote_copy(
                src_ref=src, dst_ref=dst, send_sem=ssem, recv_sem=rsem,
                device_id=dev, device_id_type=pl.DeviceIdType.MESH,
            ).start()

        def wait_recv(dst, rsem):
            pltpu.make_async_remote_copy(
                src_ref=dst, dst_ref=dst, send_sem=copy_sem, recv_sem=rsem,
                device_id=partner, device_id_type=pl.DeviceIdType.MESH,
            ).wait_recv()

        def wait_send(src, ssem):
            pltpu.make_async_remote_copy(
                src_ref=src, dst_ref=src, send_sem=ssem, recv_sem=copy_sem,
                device_id=partner, device_id_type=pl.DeviceIdType.MESH,
            ).wait_send()

        barrier_sem = pltpu.get_barrier_semaphore()
        for dev in (partner, buddy):
            pl.semaphore_signal(barrier_sem, inc=1, device_id=dev,
                                device_id_type=pl.DeviceIdType.MESH)

        @pl.when(has_right)
        def _():
            pl.semaphore_signal(barrier_sem, inc=1, device_id=right,
                                device_id_type=pl.DeviceIdType.MESH)

        @pl.when(has_left)
        def _():
            pl.semaphore_signal(barrier_sem, inc=1, device_id=left,
                                device_id_type=pl.DeviceIdType.MESH)

        n_nbrs = 2 + has_right.astype(jnp.int32) + has_left.astype(jnp.int32)
        pl.semaphore_wait(barrier_sem, n_nbrs)

        cp = pltpu.make_async_copy(
            x_ref, out_ref.at[pl.ds(my_z * m_per, m_per), :], copy_sem
        )
        cp.start()

        for s_ in range(S):
            src = x_ref.at[pl.ds(q * qrows + s_ * qseg, qseg), :]

            @pl.when(has_right)
            def _():
                send(src, qsl(my_z, q, s_), szr.at[0, s_], rzr.at[0, s_], right)

            @pl.when(has_left)
            def _():
                send(src, qsl(my_z, q, s_), szl.at[0, s_], rzl.at[0, s_], left)

        def chunk_of(d, idx):
            return my_z - 1 - idx if d == 0 else my_z + 1 + idx

        def ev_guard(d, idx):
            return idx < my_z if d == 0 else idx < NZ - 1 - my_z

        for idx in range(NZ - 1):
            for s_ in range(S):
                for d, rz, sz, fwd_guard, fwd_dev in (
                    (0, rzr, szr, has_right, right),
                    (1, rzl, szl, has_left, left),
                ):
                    @pl.when(ev_guard(d, idx))
                    def _(d=d, rz=rz, sz=sz, fwd_guard=fwd_guard,
                          fwd_dev=fwd_dev):
                        c = chunk_of(d, idx)
                        dst = qsl(c, q, s_)
                        wait_recv(dst, rz.at[idx, s_])

                        @pl.when(fwd_guard)
                        def _():
                            send(dst, dst, sz.at[idx + 1, s_],
                                 rz.at[idx + 1, s_], fwd_dev)

                        send(dst, dst, sxd.at[d, idx, s_],
                             rxd.at[d, idx, s_], partner)
                        send(dst, dst, syd.at[d, idx, s_],
                             ryd.at[d, idx, s_], buddy)

        for idx in range(NZ - 1):
            for s_ in range(S):
                for d in (0, 1):
                    @pl.when(ev_guard(d, idx))
                    def _(d=d):
                        c = chunk_of(d, idx)
                        dstp = qsl(c, qp, s_)
                        wait_recv(dstp, rxd.at[d, idx, s_])
                        if s_ % 2 == 0:
                            send(dstp, dstp, syrel.at[d, idx, s_ // 2],
                                 ryrel.at[d, idx, s_ // 2], buddy)

                    @pl.when(ev_guard(d, idx))
                    def _(d=d):
                        c = chunk_of(d, idx)
                        dstb = qsl(c, qb, s_)
                        wait_recv(dstb, ryd.at[d, idx, s_])
                        if s_ % 2 == 1:
                            send(dstb, dstb, sxrel.at[d, idx, s_ // 2],
                                 rxrel.at[d, idx, s_ // 2], partner)

        cp.wait()

        for idx in range(NZ - 1):
            for k in range(S // 2):
                for d in (0, 1):
                    @pl.when(ev_guard(d, idx))
                    def _(d=d):
                        c = chunk_of(d, idx)
                        wait_recv(qsl(c, qd, 2 * k + 1), rxrel.at[d, idx, k])
                        wait_recv(qsl(c, qd, 2 * k), ryrel.at[d, idx, k])

        dummy = x_ref.at[pl.ds(0, qseg), :]
        for s_ in range(S):
            @pl.when(has_right)
            def _():
                wait_send(dummy, szr.at[0, s_])

            @pl.when(has_left)
            def _():
                wait_send(dummy, szl.at[0, s_])

        for idx in range(NZ - 1):
            for s_ in range(S):
                for d in (0, 1):
                    g = ev_guard(d, idx)
                    fwd_guard = has_right if d == 0 else has_left
                    sz = szr if d == 0 else szl

                    @pl.when(jnp.logical_and(g, fwd_guard))
                    def _(sz=sz):
                        wait_send(dummy, sz.at[idx + 1, s_])

                    @pl.when(g)
                    def _(d=d):
                        wait_send(dummy, sxd.at[d, idx, s_])
                        wait_send(dummy, syd.at[d, idx, s_])
                        if s_ % 2 == 0:
                            wait_send(dummy, syrel.at[d, idx, s_ // 2])
                        else:
                            wait_send(dummy, sxrel.at[d, idx, s_ // 2])

    semz = pltpu.SemaphoreType.DMA((NZ - 1, S))
    semd = pltpu.SemaphoreType.DMA((2, NZ - 1, S))
    semr = pltpu.SemaphoreType.DMA((2, NZ - 1, S // 2))
    return pl.pallas_call(
        body,
        out_shape=jax.ShapeDtypeStruct((NZ * m_per, n), jnp.bfloat16),
        in_specs=[pl.BlockSpec(memory_space=pl.ANY)],
        out_specs=pl.BlockSpec(memory_space=pltpu.MemorySpace.HBM),
        scratch_shapes=[
            pltpu.SemaphoreType.DMA,
            semz, semz, semz, semz,
            semd, semd, semd, semd,
            semr, semr, semr, semr,
        ],
        compiler_params=pltpu.CompilerParams(collective_id=0),
    )(x)
